# baseline (speedup 1.0000x reference)
import functools

import jax
import jax.numpy as jnp
from jax import lax
from jax.experimental import pallas as pl
from jax.experimental.pallas import tpu as pltpu
from jax.experimental.pallas import tpu_sc as plsc

N = 10000
E = 320000
D = 128
H = 64

NUM_CORES = 2
NUM_SUBCORES = 16
NUM_WORKERS = NUM_CORES * NUM_SUBCORES

CHUNK = 128
NBUF = 2
NPHASE = 2
PCHUNKS = 40
WCHUNKS = NPHASE * PCHUNKS
EDGES_PER_W = WCHUNKS * CHUNK
E_PAD = EDGES_PER_W * NUM_WORKERS
N_ACC = 10112
INIT_ROWS = N_ACC // NUM_SUBCORES
OUT_ROWS = 624
OUT_TAIL = N - NUM_SUBCORES * OUT_ROWS


def _sc_aggregate():
    mesh = plsc.VectorSubcoreMesh(core_axis_name="c", subcore_axis_name="s")

    @functools.partial(
        pl.kernel,
        mesh=mesh,
        compiler_params=pltpu.CompilerParams(use_tc_tiling_on_sc=False),
        out_type=jax.ShapeDtypeStruct((NUM_CORES * N, D), jnp.float32),
        scratch_types=[
            pltpu.VMEM((PCHUNKS, CHUNK), jnp.int32),
            pltpu.VMEM((PCHUNKS, CHUNK), jnp.int32),
            pltpu.VMEM((NBUF, CHUNK, D), jnp.float32),
            pltpu.VMEM_SHARED((N_ACC, D), jnp.float32),
            [pltpu.SemaphoreType.DMA] * NBUF,
        ],
    )
    def sc_agg(x_hbm, src_hbm, dst_hbm, zeros_hbm, out_hbm,
               src_v, dst_v, rows_v, acc, sems):
        c = lax.axis_index("c")
        s = lax.axis_index("s")
        w = c * NUM_SUBCORES + s

        pltpu.sync_copy(zeros_hbm.at[pl.ds(s * INIT_ROWS, INIT_ROWS)],
                        acc.at[pl.ds(s * INIT_ROWS, INIT_ROWS)])
        plsc.subcore_barrier()

        def run_phase(p, carry):
            base = w * WCHUNKS + p * PCHUNKS
            pltpu.sync_copy(src_hbm.at[pl.ds(base, PCHUNKS)], src_v)
            pltpu.sync_copy(dst_hbm.at[pl.ds(base, PCHUNKS)], dst_v)

            def body(k, carry2):
                i = k * NBUF
                handles = []
                for j in range(NBUF):
                    handles.append(pltpu.async_copy(
                        x_hbm.at[src_v.at[i + j]], rows_v.at[j], sems[j]))
                for j in range(NBUF):
                    handles[j].wait()
                    pltpu.sync_copy(rows_v.at[j], acc.at[dst_v.at[i + j]],
                                    add=True)
                return carry2

            lax.fori_loop(0, PCHUNKS // NBUF, body, 0)
            return carry

        lax.fori_loop(0, NPHASE, run_phase, 0)
        plsc.subcore_barrier()

        pltpu.sync_copy(
            acc.at[pl.ds(s * OUT_ROWS, OUT_ROWS)],
            out_hbm.at[pl.ds(c * N + s * OUT_ROWS, OUT_ROWS)])

        @pl.when(s == 0)
        def _():
            tail = NUM_SUBCORES * OUT_ROWS
            pltpu.sync_copy(
                acc.at[pl.ds(tail, OUT_TAIL)],
                out_hbm.at[pl.ds(c * N + tail, OUT_TAIL)])

    return sc_agg


_SC_AGG = _sc_aggregate()


def _mlp_body(x_ref, p0_ref, p1_ref, eps_ref,
              w1_ref, b1_ref, g1_ref, bt1_ref,
              w2_ref, b2_ref, g2_ref, bt2_ref,
              w3_ref, b3_ref, out_ref):
    scale = 2.0 + eps_ref[0, 0]
    v = scale * x_ref[...] + p0_ref[...] + p1_ref[...]

    h = jnp.dot(v, w1_ref[...], preferred_element_type=jnp.float32)
    h = h + b1_ref[...]
    m = jnp.mean(h, axis=-1, keepdims=True)
    d = h - m
    var = jnp.mean(d * d, axis=-1, keepdims=True)
    h = d * lax.rsqrt(var + 1e-5) * g1_ref[...] + bt1_ref[...]
    h = jnp.maximum(h, 0.0)

    h = jnp.dot(h, w2_ref[...], preferred_element_type=jnp.float32)
    h = h + b2_ref[...]
    m = jnp.mean(h, axis=-1, keepdims=True)
    d = h - m
    var = jnp.mean(d * d, axis=-1, keepdims=True)
    h = d * lax.rsqrt(var + 1e-5) * g2_ref[...] + bt2_ref[...]
    h = jnp.maximum(h, 0.0)

    h = jnp.dot(h, w3_ref[...], preferred_element_type=jnp.float32)
    out_ref[...] = h + b3_ref[...]


def _run_mlp(x, parts, eps, W1, b1, g1, bt1, W2, b2, g2, bt2, W3, b3):
    rows = 1000
    nblk = N // rows
    grid = (nblk,)
    row_spec = pl.BlockSpec((rows, D), lambda i: (i, 0))
    p0_spec = pl.BlockSpec((rows, D), lambda i: (i, 0))
    p1_spec = pl.BlockSpec((rows, D), lambda i: (nblk + i, 0))

    def full(shape):
        return pl.BlockSpec(shape, lambda i: tuple(0 for _ in shape))

    return pl.pallas_call(
        _mlp_body,
        grid=grid,
        in_specs=[
            row_spec, p0_spec, p1_spec,
            pl.BlockSpec(memory_space=pltpu.SMEM),
            full((D, H)), full((1, H)), full((1, H)), full((1, H)),
            full((H, H)), full((1, H)), full((1, H)), full((1, H)),
            full((H, D)), full((1, D)),
        ],
        out_specs=row_spec,
        out_shape=jax.ShapeDtypeStruct((N, D), jnp.float32),
    )(x, parts, parts, eps.reshape(1, 1),
      W1, b1.reshape(1, H), g1.reshape(1, H), bt1.reshape(1, H),
      W2, b2.reshape(1, H), g2.reshape(1, H), bt2.reshape(1, H),
      W3, b3.reshape(1, D))


def kernel(x, edge_index, eps, W1, b1, g1, bt1, W2, b2, g2, bt2, W3, b3):
    src = edge_index[0]
    dst = edge_index[1]
    pad = E_PAD - E
    src_p = jnp.concatenate([src, jnp.zeros((pad,), jnp.int32)])
    dst_p = jnp.concatenate([dst, jnp.full((pad,), N, jnp.int32)])
    src_t = src_p.reshape(NUM_WORKERS * WCHUNKS, CHUNK)
    dst_t = dst_p.reshape(NUM_WORKERS * WCHUNKS, CHUNK)
    zeros = jnp.zeros((N_ACC, D), jnp.float32)

    parts = _SC_AGG(x, src_t, dst_t, zeros)

    return _run_mlp(x, parts, eps,
                    W1, b1, g1, bt1, W2, b2, g2, bt2, W3, b3)


# speedup vs baseline: 1.4581x; 1.4581x over previous
import functools

import jax
import jax.numpy as jnp
from jax import lax
from jax.experimental import pallas as pl
from jax.experimental.pallas import tpu as pltpu
from jax.experimental.pallas import tpu_sc as plsc

N = 10000
E = 320000
D = 128
H = 64
HD = D // 2

NUM_CORES = 2
NUM_SUBCORES = 16

CHUNK = 128
NBUF = 5
TCHUNKS = 160
NGROUPS = TCHUNKS // NBUF
EDGES_PER_TILE = TCHUNKS * CHUNK
E_PAD = EDGES_PER_TILE * NUM_SUBCORES
N_ACC = 10112
INIT_ROWS = N_ACC // NUM_SUBCORES
OUT_ROWS = 624
OUT_TAIL = N - NUM_SUBCORES * OUT_ROWS


def _sc_aggregate():
    mesh = plsc.VectorSubcoreMesh(core_axis_name="c", subcore_axis_name="s")

    @functools.partial(
        pl.kernel,
        mesh=mesh,
        compiler_params=pltpu.CompilerParams(use_tc_tiling_on_sc=False),
        out_type=jax.ShapeDtypeStruct((NUM_CORES * N, HD), jnp.float32),
        scratch_types=[
            pltpu.VMEM((TCHUNKS, CHUNK), jnp.int32),
            pltpu.VMEM((TCHUNKS, CHUNK), jnp.int32),
            pltpu.VMEM((NBUF, CHUNK, HD), jnp.float32),
            pltpu.VMEM_SHARED((N_ACC, HD), jnp.float32),
            [pltpu.SemaphoreType.DMA] * NBUF,
            [pltpu.SemaphoreType.DMA] * NBUF,
        ],
    )
    def sc_agg(xh_hbm, src_hbm, dst_hbm, zeros_hbm, out_hbm,
               src_v, dst_v, rows_v, acc, gsems, ssems):
        c = lax.axis_index("c")
        s = lax.axis_index("s")

        w = c * NUM_SUBCORES + s
        pltpu.sync_copy(src_hbm.at[pl.ds(w * TCHUNKS, TCHUNKS)], src_v)
        pltpu.sync_copy(dst_hbm.at[pl.ds(s * TCHUNKS, TCHUNKS)], dst_v)
        pltpu.sync_copy(zeros_hbm.at[pl.ds(s * INIT_ROWS, INIT_ROWS)],
                        acc.at[pl.ds(s * INIT_ROWS, INIT_ROWS)])
        plsc.subcore_barrier()

        def fire_gather(i, b):
            pltpu.async_copy(xh_hbm.at[src_v.at[i]], rows_v.at[b], gsems[b])

        def fire_scatter(i, b):
            pltpu.async_copy(rows_v.at[b], acc.at[dst_v.at[i]], ssems[b],
                             add=True)

        def wait_gather(b):
            pltpu.make_async_copy(
                xh_hbm.at[pl.ds(0, CHUNK)], rows_v.at[b], gsems[b]).wait()

        def wait_scatter(b):
            pltpu.make_async_copy(
                rows_v.at[b], acc.at[pl.ds(0, CHUNK)], ssems[b]).wait()

        for b in range(NBUF):
            fire_gather(b, b)

        def body(g, carry):
            i0 = g * NBUF
            for b in range(NBUF):
                wait_gather(b)
                fire_scatter(i0 + b, b)
            for b in range(NBUF):
                wait_scatter(b)
                fire_gather(i0 + NBUF + b, b)
            return carry

        lax.fori_loop(0, NGROUPS - 1, body, 0)

        i0 = (NGROUPS - 1) * NBUF
        for b in range(NBUF):
            wait_gather(b)
            fire_scatter(i0 + b, b)
        for b in range(NBUF):
            wait_scatter(b)

        plsc.subcore_barrier()

        pltpu.sync_copy(
            acc.at[pl.ds(s * OUT_ROWS, OUT_ROWS)],
            out_hbm.at[pl.ds(c * N + s * OUT_ROWS, OUT_ROWS)])

        @pl.when(s == 0)
        def _():
            tail = NUM_SUBCORES * OUT_ROWS
            pltpu.sync_copy(
                acc.at[pl.ds(tail, OUT_TAIL)],
                out_hbm.at[pl.ds(c * N + tail, OUT_TAIL)])

    return sc_agg


_SC_AGG = _sc_aggregate()


def _mlp_body(x_ref, pl_ref, pr_ref, eps_ref,
              w1_ref, b1_ref, g1_ref, bt1_ref,
              w2_ref, b2_ref, g2_ref, bt2_ref,
              w3_ref, b3_ref, out_ref):
    scale = 2.0 + eps_ref[0, 0]
    agg = jnp.concatenate([pl_ref[...], pr_ref[...]], axis=-1)
    v = scale * x_ref[...] + agg

    h = jnp.dot(v, w1_ref[...], preferred_element_type=jnp.float32)
    h = h + b1_ref[...]
    m = jnp.mean(h, axis=-1, keepdims=True)
    d = h - m
    var = jnp.mean(d * d, axis=-1, keepdims=True)
    h = d * lax.rsqrt(var + 1e-5) * g1_ref[...] + bt1_ref[...]
    h = jnp.maximum(h, 0.0)

    h = jnp.dot(h, w2_ref[...], preferred_element_type=jnp.float32)
    h = h + b2_ref[...]
    m = jnp.mean(h, axis=-1, keepdims=True)
    d = h - m
    var = jnp.mean(d * d, axis=-1, keepdims=True)
    h = d * lax.rsqrt(var + 1e-5) * g2_ref[...] + bt2_ref[...]
    h = jnp.maximum(h, 0.0)

    h = jnp.dot(h, w3_ref[...], preferred_element_type=jnp.float32)
    out_ref[...] = h + b3_ref[...]


def _run_mlp(x, parts, eps, W1, b1, g1, bt1, W2, b2, g2, bt2, W3, b3):
    rows = 1000
    nblk = N // rows
    grid = (nblk,)
    row_spec = pl.BlockSpec((rows, D), lambda i: (i, 0))
    left_spec = pl.BlockSpec((rows, HD), lambda i: (i, 0))
    right_spec = pl.BlockSpec((rows, HD), lambda i: (nblk + i, 0))

    def full(shape):
        return pl.BlockSpec(shape, lambda i: tuple(0 for _ in shape))

    return pl.pallas_call(
        _mlp_body,
        grid=grid,
        in_specs=[
            row_spec, left_spec, right_spec,
            pl.BlockSpec(memory_space=pltpu.SMEM),
            full((D, H)), full((1, H)), full((1, H)), full((1, H)),
            full((H, H)), full((1, H)), full((1, H)), full((1, H)),
            full((H, D)), full((1, D)),
        ],
        out_specs=row_spec,
        out_shape=jax.ShapeDtypeStruct((N, D), jnp.float32),
    )(x, parts, parts, eps.reshape(1, 1),
      W1, b1.reshape(1, H), g1.reshape(1, H), bt1.reshape(1, H),
      W2, b2.reshape(1, H), g2.reshape(1, H), bt2.reshape(1, H),
      W3, b3.reshape(1, D))


def kernel(x, edge_index, eps, W1, b1, g1, bt1, W2, b2, g2, bt2, W3, b3):
    src = edge_index[0]
    dst = edge_index[1]
    pad = E_PAD - E
    src_p = jnp.concatenate([src, jnp.zeros((pad,), jnp.int32)])
    dst_p = jnp.concatenate([dst, jnp.full((pad,), N, jnp.int32)])
    src_t = src_p.reshape(NUM_SUBCORES * TCHUNKS, CHUNK)
    src2 = jnp.concatenate([src_t, src_t + N], axis=0)
    dst_t = dst_p.reshape(NUM_SUBCORES * TCHUNKS, CHUNK)
    xh = jnp.concatenate([x[:, :HD], x[:, HD:]], axis=0)
    zeros = jnp.zeros((N_ACC, HD), jnp.float32)

    parts = _SC_AGG(xh, src2, dst_t, zeros)

    return _run_mlp(x, parts, eps,
                    W1, b1, g1, bt1, W2, b2, g2, bt2, W3, b3)


# speedup vs baseline: 2.1804x; 1.4954x over previous
import functools

import jax
import jax.numpy as jnp
from jax import lax
from jax.experimental import pallas as pl
from jax.experimental.pallas import tpu as pltpu
from jax.experimental.pallas import tpu_sc as plsc

N = 10000
E = 320000
D = 128
H = 64
HD = D // 2

NUM_CORES = 2
NUM_SUBCORES = 16

CHUNK = 128
NBUF = 4
TCHUNKS = 160
NPHASE = 4
PCHUNKS = TCHUNKS // NPHASE
PGROUPS = PCHUNKS // NBUF
EDGES_PER_TILE = TCHUNKS * CHUNK
E_PAD = EDGES_PER_TILE * NUM_SUBCORES
N_ACC = 10112
INIT_ROWS = N_ACC // NUM_SUBCORES
OUT_ROWS = 624
OUT_TAIL = N - NUM_SUBCORES * OUT_ROWS


def _sc_aggregate():
    mesh = plsc.VectorSubcoreMesh(core_axis_name="c", subcore_axis_name="s")

    @functools.partial(
        pl.kernel,
        mesh=mesh,
        compiler_params=pltpu.CompilerParams(use_tc_tiling_on_sc=False),
        out_type=jax.ShapeDtypeStruct((NUM_CORES * N, HD), jnp.float32),
        scratch_types=[
            pltpu.VMEM((PCHUNKS, CHUNK), jnp.int32),
            pltpu.VMEM((PCHUNKS, CHUNK), jnp.int32),
            pltpu.VMEM((NBUF, CHUNK, HD), jnp.float32),
            pltpu.VMEM_SHARED((N, HD), jnp.float32),
            pltpu.VMEM_SHARED((N_ACC, HD), jnp.float32),
            [pltpu.SemaphoreType.DMA] * NBUF,
            [pltpu.SemaphoreType.DMA] * NBUF,
        ],
    )
    def sc_agg(xh_hbm, src_hbm, dst_hbm, zeros_hbm, out_hbm,
               src_v, dst_v, rows_v, xs, acc, gsems, ssems):
        c = lax.axis_index("c")
        s = lax.axis_index("s")

        pltpu.sync_copy(xh_hbm.at[pl.ds(c * N + s * OUT_ROWS, OUT_ROWS)],
                        xs.at[pl.ds(s * OUT_ROWS, OUT_ROWS)])
        pltpu.sync_copy(zeros_hbm.at[pl.ds(s * INIT_ROWS, INIT_ROWS)],
                        acc.at[pl.ds(s * INIT_ROWS, INIT_ROWS)])

        @pl.when(s == 0)
        def _():
            tail = NUM_SUBCORES * OUT_ROWS
            pltpu.sync_copy(xh_hbm.at[pl.ds(c * N + tail, OUT_TAIL)],
                            xs.at[pl.ds(tail, OUT_TAIL)])

        plsc.subcore_barrier()

        def fire_gather(i, b):
            pltpu.async_copy(xs.at[src_v.at[i]], rows_v.at[b], gsems[b])

        def fire_scatter(i, b):
            pltpu.async_copy(rows_v.at[b], acc.at[dst_v.at[i]], ssems[b],
                             add=True)

        def wait_gather(b):
            pltpu.make_async_copy(
                xh_hbm.at[pl.ds(0, CHUNK)], rows_v.at[b], gsems[b]).wait()

        def wait_scatter(b):
            pltpu.make_async_copy(
                rows_v.at[b], acc.at[pl.ds(0, CHUNK)], ssems[b]).wait()

        def run_phase(ph, carry):
            base = s * TCHUNKS + ph * PCHUNKS
            pltpu.sync_copy(src_hbm.at[pl.ds(base, PCHUNKS)], src_v)
            pltpu.sync_copy(dst_hbm.at[pl.ds(base, PCHUNKS)], dst_v)

            for b in range(NBUF):
                fire_gather(b, b)

            def body(g, carry2):
                i0 = g * NBUF
                for b in range(NBUF):
                    wait_gather(b)
                    fire_scatter(i0 + b, b)
                for b in range(NBUF):
                    wait_scatter(b)
                    fire_gather(i0 + NBUF + b, b)
                return carry2

            lax.fori_loop(0, PGROUPS - 1, body, 0)

            i0 = (PGROUPS - 1) * NBUF
            for b in range(NBUF):
                wait_gather(b)
                fire_scatter(i0 + b, b)
            for b in range(NBUF):
                wait_scatter(b)
            return carry

        lax.fori_loop(0, NPHASE, run_phase, 0)
        plsc.subcore_barrier()

        pltpu.sync_copy(
            acc.at[pl.ds(s * OUT_ROWS, OUT_ROWS)],
            out_hbm.at[pl.ds(c * N + s * OUT_ROWS, OUT_ROWS)])

        @pl.when(s == 0)
        def _():
            tail = NUM_SUBCORES * OUT_ROWS
            pltpu.sync_copy(
                acc.at[pl.ds(tail, OUT_TAIL)],
                out_hbm.at[pl.ds(c * N + tail, OUT_TAIL)])

    return sc_agg


_SC_AGG = _sc_aggregate()


def _mlp_body(x_ref, pl_ref, pr_ref, eps_ref,
              w1_ref, b1_ref, g1_ref, bt1_ref,
              w2_ref, b2_ref, g2_ref, bt2_ref,
              w3_ref, b3_ref, out_ref):
    scale = 2.0 + eps_ref[0, 0]
    agg = jnp.concatenate([pl_ref[...], pr_ref[...]], axis=-1)
    v = scale * x_ref[...] + agg

    h = jnp.dot(v, w1_ref[...], preferred_element_type=jnp.float32)
    h = h + b1_ref[...]
    m = jnp.mean(h, axis=-1, keepdims=True)
    d = h - m
    var = jnp.mean(d * d, axis=-1, keepdims=True)
    h = d * lax.rsqrt(var + 1e-5) * g1_ref[...] + bt1_ref[...]
    h = jnp.maximum(h, 0.0)

    h = jnp.dot(h, w2_ref[...], preferred_element_type=jnp.float32)
    h = h + b2_ref[...]
    m = jnp.mean(h, axis=-1, keepdims=True)
    d = h - m
    var = jnp.mean(d * d, axis=-1, keepdims=True)
    h = d * lax.rsqrt(var + 1e-5) * g2_ref[...] + bt2_ref[...]
    h = jnp.maximum(h, 0.0)

    h = jnp.dot(h, w3_ref[...], preferred_element_type=jnp.float32)
    out_ref[...] = h + b3_ref[...]


def _run_mlp(x, parts, eps, W1, b1, g1, bt1, W2, b2, g2, bt2, W3, b3):
    rows = 1000
    nblk = N // rows
    grid = (nblk,)
    row_spec = pl.BlockSpec((rows, D), lambda i: (i, 0))
    left_spec = pl.BlockSpec((rows, HD), lambda i: (i, 0))
    right_spec = pl.BlockSpec((rows, HD), lambda i: (nblk + i, 0))

    def full(shape):
        return pl.BlockSpec(shape, lambda i: tuple(0 for _ in shape))

    return pl.pallas_call(
        _mlp_body,
        grid=grid,
        in_specs=[
            row_spec, left_spec, right_spec,
            pl.BlockSpec(memory_space=pltpu.SMEM),
            full((D, H)), full((1, H)), full((1, H)), full((1, H)),
            full((H, H)), full((1, H)), full((1, H)), full((1, H)),
            full((H, D)), full((1, D)),
        ],
        out_specs=row_spec,
        out_shape=jax.ShapeDtypeStruct((N, D), jnp.float32),
    )(x, parts, parts, eps.reshape(1, 1),
      W1, b1.reshape(1, H), g1.reshape(1, H), bt1.reshape(1, H),
      W2, b2.reshape(1, H), g2.reshape(1, H), bt2.reshape(1, H),
      W3, b3.reshape(1, D))


def kernel(x, edge_index, eps, W1, b1, g1, bt1, W2, b2, g2, bt2, W3, b3):
    src = edge_index[0]
    dst = edge_index[1]
    pad = E_PAD - E
    src_p = jnp.concatenate([src, jnp.zeros((pad,), jnp.int32)])
    dst_p = jnp.concatenate([dst, jnp.full((pad,), N, jnp.int32)])
    src_t = src_p.reshape(NUM_SUBCORES * TCHUNKS, CHUNK)
    dst_t = dst_p.reshape(NUM_SUBCORES * TCHUNKS, CHUNK)
    xh = jnp.concatenate([x[:, :HD], x[:, HD:]], axis=0)
    zeros = jnp.zeros((N_ACC, HD), jnp.float32)

    parts = _SC_AGG(xh, src_t, dst_t, zeros)

    return _run_mlp(x, parts, eps,
                    W1, b1, g1, bt1, W2, b2, g2, bt2, W3, b3)
